# baseline (speedup 1.0000x reference)
import functools

import jax
import jax.numpy as jnp
from jax import lax
from jax.experimental import pallas as pl
from jax.experimental.pallas import tpu as pltpu
from jax.experimental.pallas import tpu_sc as plsc

_BATCH = 16384
_HID = 1024
_ROWS = 1001
_TAB_WORDS = _ROWS * _HID
_NC = 2
_NS = 16
_NW = _NC * _NS
_B_PER_W = _BATCH // _NW
_CHUNK = 32
_NCHUNK = _B_PER_W // _CHUNK
_NBUF = 2


def _make_gather():
    mesh = plsc.VectorSubcoreMesh(core_axis_name="c", subcore_axis_name="s")

    scratch = [pltpu.VMEM_SHARED((_TAB_WORDS,), jnp.float32)]
    scratch += [pltpu.VMEM((_B_PER_W,), jnp.int32)]
    scratch += [pltpu.VMEM((_CHUNK * _HID,), jnp.float32) for _ in range(_NBUF)]
    scratch += [pltpu.SemaphoreType.DMA for _ in range(2 * _NBUF)]

    @functools.partial(
        pl.kernel,
        mesh=mesh,
        out_type=jax.ShapeDtypeStruct((_BATCH * _HID,), jnp.float32),
        scratch_types=scratch,
    )
    def gather_kernel(table_hbm, idx_hbm, out_hbm, tab_sh, idx_v, *bufs):
        rows = bufs[:_NBUF]
        gsem = bufs[_NBUF:2 * _NBUF]
        wsem = bufs[2 * _NBUF:]
        s = lax.axis_index("s")
        wid = s * _NC + lax.axis_index("c")
        base = wid * _B_PER_W
        pltpu.sync_copy(idx_hbm.at[pl.ds(base, _B_PER_W)], idx_v)

        stripe = _TAB_WORDS // _NS
        piece = stripe // 2
        soff = pl.multiple_of(s * stripe, 8)
        hin = []
        for p in range(2):
            hin.append(pltpu.async_copy(
                table_hbm.at[pl.ds(soff + p * piece, piece)],
                rows[p].at[pl.ds(0, piece)], gsem[p]))
        hout = []
        for p in range(2):
            hin[p].wait()
            hout.append(pltpu.async_copy(
                rows[p].at[pl.ds(0, piece)],
                tab_sh.at[pl.ds(soff + p * piece, piece)], wsem[p]))
        for p in range(2):
            hout[p].wait()
        plsc.subcore_barrier()

        def fire_chunk(c):
            b = c % _NBUF
            handles = []
            for v in range(_CHUNK // 16):
                iv = idx_v[pl.ds(c * _CHUNK + v * 16, 16)]
                for j in range(16):
                    woff = pl.multiple_of(iv[j] * _HID, 8)
                    handles.append(pltpu.async_copy(
                        tab_sh.at[pl.ds(woff, _HID)],
                        rows[b].at[pl.ds((v * 16 + j) * _HID, _HID)],
                        gsem[b],
                    ))
            return handles

        def writeback(c):
            b = c % _NBUF
            return pltpu.async_copy(
                rows[b],
                out_hbm.at[pl.ds((base + c * _CHUNK) * _HID, _CHUNK * _HID)],
                wsem[b],
            )

        g = {}
        w = {}
        g[0] = fire_chunk(0)
        for c in range(_NCHUNK):
            if c + 1 < _NCHUNK:
                if c - 1 in w:
                    w.pop(c - 1).wait()
                g[c + 1] = fire_chunk(c + 1)
            for h in g.pop(c):
                h.wait()
            w[c] = writeback(c)
        for c in sorted(w):
            w.pop(c).wait()

    return gather_kernel


_gather = _make_gather()


@jax.jit
def kernel(x, condition, timestep, time_embed):
    flat = _gather(time_embed.reshape(-1), timestep)
    return flat.reshape(_BATCH, _HID)


# speedup vs baseline: 1.0168x; 1.0168x over previous
import functools

import jax
import jax.numpy as jnp
from jax import lax
from jax.experimental import pallas as pl
from jax.experimental.pallas import tpu as pltpu
from jax.experimental.pallas import tpu_sc as plsc

_BATCH = 16384
_HID = 1024
_ROWS = 1001
_TAB_WORDS = _ROWS * _HID
_NC = 2
_NS = 16
_NW = _NC * _NS
_B_PER_W = _BATCH // _NW
_CHUNK = 32
_NCHUNK = _B_PER_W // _CHUNK
_NBUF = 2


def _make_gather():
    mesh = plsc.VectorSubcoreMesh(core_axis_name="c", subcore_axis_name="s")

    scratch = [pltpu.VMEM_SHARED((_TAB_WORDS,), jnp.float32)]
    scratch += [pltpu.VMEM((_B_PER_W,), jnp.int32)]
    scratch += [pltpu.VMEM((_CHUNK * _HID,), jnp.float32) for _ in range(_NBUF)]
    scratch += [pltpu.SemaphoreType.DMA for _ in range(2 * _NBUF)]

    @functools.partial(
        pl.kernel,
        mesh=mesh,
        out_type=jax.ShapeDtypeStruct((_BATCH * _HID,), jnp.float32),
        scratch_types=scratch,
    )
    def gather_kernel(table_hbm, idx_hbm, out_hbm, tab_sh, idx_v, *bufs):
        rows = bufs[:_NBUF]
        gsem = bufs[_NBUF:2 * _NBUF]
        wsem = bufs[2 * _NBUF:]
        s = lax.axis_index("s")
        wid = s * _NC + lax.axis_index("c")
        base = wid * _B_PER_W
        pltpu.sync_copy(idx_hbm.at[pl.ds(base, _B_PER_W)], idx_v)

        stripe = _TAB_WORDS // _NS
        piece = stripe // 2
        soff = pl.multiple_of(s * stripe, 8)
        hin = []
        for p in range(2):
            hin.append(pltpu.async_copy(
                table_hbm.at[pl.ds(soff + p * piece, piece)],
                rows[p].at[pl.ds(0, piece)], gsem[p]))
        hout = []
        for p in range(2):
            hin[p].wait()
            hout.append(pltpu.async_copy(
                rows[p].at[pl.ds(0, piece)],
                tab_sh.at[pl.ds(soff + p * piece, piece)], wsem[p]))
        for p in range(2):
            hout[p].wait()
        plsc.subcore_barrier()

        def fire_chunk(c):
            b = c % _NBUF
            for v in range(_CHUNK // 16):
                iv = idx_v[pl.ds(c * _CHUNK + v * 16, 16)]
                for j in range(16):
                    woff = pl.multiple_of(iv[j] * _HID, 8)
                    pltpu.async_copy(
                        tab_sh.at[pl.ds(woff, _HID)],
                        rows[b].at[pl.ds((v * 16 + j) * _HID, _HID)],
                        gsem[b],
                    )
            return pltpu.make_async_copy(
                table_hbm.at[pl.ds(0, _CHUNK * _HID)], rows[b], gsem[b])

        def writeback(c):
            b = c % _NBUF
            return pltpu.async_copy(
                rows[b],
                out_hbm.at[pl.ds((base + c * _CHUNK) * _HID, _CHUNK * _HID)],
                wsem[b],
            )

        g = {}
        w = {}
        g[0] = fire_chunk(0)
        for c in range(_NCHUNK):
            if c + 1 < _NCHUNK:
                if c - 1 in w:
                    w.pop(c - 1).wait()
                g[c + 1] = fire_chunk(c + 1)
            g.pop(c).wait()
            w[c] = writeback(c)
        for c in sorted(w):
            w.pop(c).wait()

    return gather_kernel


_gather = _make_gather()


@jax.jit
def kernel(x, condition, timestep, time_embed):
    flat = _gather(time_embed.reshape(-1), timestep)
    return flat.reshape(_BATCH, _HID)


# speedup vs baseline: 1.0688x; 1.0512x over previous
import functools

import jax
import jax.numpy as jnp
from jax import lax
from jax.experimental import pallas as pl
from jax.experimental.pallas import tpu as pltpu
from jax.experimental.pallas import tpu_sc as plsc

_BATCH = 16384
_HID = 1024
_ROWS = 1001
_NC = 2
_NS = 16
_NW = _NC * _NS
_B_PER_W = _BATCH // _NW
_CHUNK = 16
_NCHUNK = _B_PER_W // _CHUNK


def _make_gather():
    mesh = plsc.VectorSubcoreMesh(core_axis_name="c", subcore_axis_name="s")

    scratch = [pltpu.VMEM_SHARED((_ROWS, 8, 128), jnp.float32)]
    scratch += [pltpu.VMEM((_B_PER_W,), jnp.int32)]
    scratch += [pltpu.VMEM((_CHUNK, 8, 128), jnp.float32) for _ in range(4)]
    scratch += [pltpu.SemaphoreType.DMA for _ in range(8)]

    @functools.partial(
        pl.kernel,
        mesh=mesh,
        out_type=jax.ShapeDtypeStruct((_BATCH, 8, 128), jnp.float32),
        scratch_types=scratch,
    )
    def gather_kernel(table3d, idx_hbm, out_hbm, tab_sh, idx_v,
                      hbuf0, hbuf1, sbuf0, sbuf1, *sems):
        hbuf = (hbuf0, hbuf1)
        sbuf = (sbuf0, sbuf1)
        hgsem = sems[0:2]
        sgsem = sems[2:4]
        hwsem = sems[4:6]
        swsem = sems[6:8]
        s = lax.axis_index("s")
        wid = s * _NC + lax.axis_index("c")
        base = wid * _B_PER_W
        pltpu.sync_copy(idx_hbm.at[pl.ds(base, _B_PER_W)], idx_v)

        pend = None
        for p in range(4):
            rows = 16 if p < 3 else 14
            roff = s * 62 + p * 16
            h = pltpu.async_copy(table3d.at[pl.ds(roff, rows)],
                                 sbuf[p % 2].at[pl.ds(0, rows)], sgsem[p % 2])
            if pend is not None:
                pend.wait()
            h.wait()
            pend = pltpu.async_copy(sbuf[p % 2].at[pl.ds(0, rows)],
                                    tab_sh.at[pl.ds(roff, rows)], swsem[p % 2])
        pend.wait()

        @pl.when(s == 0)
        def _():
            pltpu.async_copy(table3d.at[pl.ds(992, 9)],
                             sbuf[0].at[pl.ds(0, 9)], sgsem[0]).wait()
            pltpu.async_copy(sbuf[0].at[pl.ds(0, 9)],
                             tab_sh.at[pl.ds(992, 9)], swsem[0]).wait()

        plsc.subcore_barrier()

        def fire(c):
            slot = (c // 2) % 2
            if c % 2 == 0:
                return pltpu.async_copy(
                    table3d.at[idx_v.at[pl.ds(c * _CHUNK, _CHUNK)]],
                    hbuf[slot], hgsem[slot])
            iv = idx_v[pl.ds(c * _CHUNK, 16)]
            for j in range(16):
                pltpu.async_copy(
                    tab_sh.at[pl.ds(iv[j], 1)],
                    sbuf[slot].at[pl.ds(j, 1)], sgsem[slot])
            return pltpu.make_async_copy(
                table3d.at[pl.ds(0, _CHUNK)], sbuf[slot], sgsem[slot])

        def writeback(c):
            slot = (c // 2) % 2
            dst = out_hbm.at[pl.ds(base + c * _CHUNK, _CHUNK)]
            if c % 2 == 0:
                return pltpu.async_copy(hbuf[slot], dst, hwsem[slot])
            return pltpu.async_copy(sbuf[slot], dst, swsem[slot])

        g = {}
        w = {}
        g[0] = fire(0)
        g[1] = fire(1)
        for c in range(_NCHUNK):
            if c + 2 < _NCHUNK:
                if c - 2 in w:
                    w.pop(c - 2).wait()
                g[c + 2] = fire(c + 2)
            g.pop(c).wait()
            w[c] = writeback(c)
        for c in sorted(w):
            w.pop(c).wait()

    return gather_kernel


_gather = _make_gather()


@jax.jit
def kernel(x, condition, timestep, time_embed):
    out3d = _gather(time_embed.reshape(_ROWS, 8, 128), timestep)
    return out3d.reshape(_BATCH, _HID)


# speedup vs baseline: 1.7985x; 1.6827x over previous
import functools

import jax
import jax.numpy as jnp
from jax import lax
from jax.experimental import pallas as pl
from jax.experimental.pallas import tpu as pltpu
from jax.experimental.pallas import tpu_sc as plsc

_BATCH = 16384
_HID = 1024
_NC = 2
_NS = 16
_NW = _NC * _NS
_B_PER_W = _BATCH // _NW
_CHUNK = 16
_NCHUNK = _B_PER_W // _CHUNK
_NBUF = 6


def _make_gather():
    mesh = plsc.VectorSubcoreMesh(core_axis_name="c", subcore_axis_name="s")

    scratch = [pltpu.VMEM((_B_PER_W,), jnp.int32)]
    scratch += [pltpu.VMEM((_CHUNK, _HID), jnp.float32) for _ in range(_NBUF)]
    scratch += [pltpu.SemaphoreType.DMA for _ in range(2 * _NBUF)]

    @functools.partial(
        pl.kernel,
        mesh=mesh,
        out_type=jax.ShapeDtypeStruct((_BATCH, _HID), jnp.float32),
        scratch_types=scratch,
    )
    def gather_kernel(table_hbm, idx_hbm, out_hbm, idx_v, *bufs):
        rows = bufs[:_NBUF]
        gsem = bufs[_NBUF:2 * _NBUF]
        wsem = bufs[2 * _NBUF:]
        wid = lax.axis_index("s") * _NC + lax.axis_index("c")
        base = wid * _B_PER_W
        pltpu.sync_copy(idx_hbm.at[pl.ds(base, _B_PER_W)], idx_v)

        def gather(c):
            b = c % _NBUF
            return pltpu.async_copy(
                table_hbm.at[idx_v.at[pl.ds(c * _CHUNK, _CHUNK)]],
                rows[b], gsem[b],
            )

        def writeback(c):
            b = c % _NBUF
            return pltpu.async_copy(
                rows[b], out_hbm.at[pl.ds(base + c * _CHUNK, _CHUNK)], wsem[b]
            )

        g = {}
        w = {}
        for c in range(_NBUF - 1):
            g[c] = gather(c)
        for c in range(_NCHUNK):
            if c + _NBUF - 1 < _NCHUNK:
                if c - 1 in w:
                    w.pop(c - 1).wait()
                g[c + _NBUF - 1] = gather(c + _NBUF - 1)
            g.pop(c).wait()
            w[c] = writeback(c)
        for c in sorted(w):
            w.pop(c).wait()

    return gather_kernel


_gather = _make_gather()


@jax.jit
def kernel(x, condition, timestep, time_embed):
    return _gather(time_embed, timestep)


# speedup vs baseline: 1.8204x; 1.0122x over previous
import functools

import jax
import jax.numpy as jnp
from jax import lax
from jax.experimental import pallas as pl
from jax.experimental.pallas import tpu as pltpu
from jax.experimental.pallas import tpu_sc as plsc

_BATCH = 16384
_HID = 1024
_NC = 2
_NS = 16
_NW = _NC * _NS
_B_PER_W = _BATCH // _NW
_CHUNK = 32
_NCHUNK = _B_PER_W // _CHUNK
_NBUF = 3


def _make_gather():
    mesh = plsc.VectorSubcoreMesh(core_axis_name="c", subcore_axis_name="s")

    scratch = [pltpu.VMEM((_B_PER_W,), jnp.int32)]
    scratch += [pltpu.VMEM((_CHUNK, _HID), jnp.float32) for _ in range(_NBUF)]
    scratch += [pltpu.SemaphoreType.DMA for _ in range(2 * _NBUF)]

    @functools.partial(
        pl.kernel,
        mesh=mesh,
        out_type=jax.ShapeDtypeStruct((_BATCH, _HID), jnp.float32),
        scratch_types=scratch,
    )
    def gather_kernel(table_hbm, idx_hbm, out_hbm, idx_v, *bufs):
        rows = bufs[:_NBUF]
        gsem = bufs[_NBUF:2 * _NBUF]
        wsem = bufs[2 * _NBUF:]
        wid = lax.axis_index("s") * _NC + lax.axis_index("c")
        base = wid * _B_PER_W
        pltpu.sync_copy(idx_hbm.at[pl.ds(base, _B_PER_W)], idx_v)

        def gather(c):
            b = c % _NBUF
            return pltpu.async_copy(
                table_hbm.at[idx_v.at[pl.ds(c * _CHUNK, _CHUNK)]],
                rows[b], gsem[b],
            )

        def writeback(c):
            b = c % _NBUF
            return pltpu.async_copy(
                rows[b], out_hbm.at[pl.ds(base + c * _CHUNK, _CHUNK)], wsem[b]
            )

        g = {}
        w = {}
        for c in range(_NBUF - 1):
            g[c] = gather(c)
        for c in range(_NCHUNK):
            if c + _NBUF - 1 < _NCHUNK:
                if c - 1 in w:
                    w.pop(c - 1).wait()
                g[c + _NBUF - 1] = gather(c + _NBUF - 1)
            g.pop(c).wait()
            w[c] = writeback(c)
        for c in sorted(w):
            w.pop(c).wait()

    return gather_kernel


_gather = _make_gather()


@jax.jit
def kernel(x, condition, timestep, time_embed):
    return _gather(time_embed, timestep)
